# baseline (speedup 1.0000x reference)
import jax
import jax.numpy as jnp
from jax.experimental import pallas as pl
from jax.experimental.pallas import tpu as pltpu

TOKENS = 8192
IN_CHANNELS = 2048
NUM_EXPERTS = 64
TILE_M = 1024


def _gate_softmax_kernel(x_ref, w_ref, b_ref, o_ref):
    logits = jax.lax.dot_general(
        x_ref[...], w_ref[...], (((1,), (1,)), ((), ())),
        preferred_element_type=jnp.float32) + b_ref[...][None, :]
    m = jnp.max(logits, axis=1, keepdims=True)
    e = jnp.exp(logits - m)
    o_ref[...] = e / jnp.sum(e, axis=1, keepdims=True)


def kernel(x, W, b):
    grid = (TOKENS // TILE_M,)
    return pl.pallas_call(
        _gate_softmax_kernel,
        grid=grid,
        in_specs=[
            pl.BlockSpec((TILE_M, IN_CHANNELS), lambda i: (i, 0)),
            pl.BlockSpec((NUM_EXPERTS, IN_CHANNELS), lambda i: (0, 0)),
            pl.BlockSpec((NUM_EXPERTS,), lambda i: (0,)),
        ],
        out_specs=pl.BlockSpec((TILE_M, NUM_EXPERTS), lambda i: (i, 0)),
        out_shape=jax.ShapeDtypeStruct((TOKENS, NUM_EXPERTS), jnp.float32),
        compiler_params=pltpu.CompilerParams(
            dimension_semantics=("parallel",),
        ),
    )(x, W, b)


# speedup vs baseline: 1.0065x; 1.0065x over previous
import jax
import jax.numpy as jnp
from jax.experimental import pallas as pl
from jax.experimental.pallas import tpu as pltpu

TOKENS = 8192
IN_CHANNELS = 2048
NUM_EXPERTS = 64
TILE_M = 1024


SUB = 2
SUB_M = TILE_M // SUB


def _gate_softmax_kernel(x_ref, w_ref, b_ref, o_ref):
    w = w_ref[...]
    bias = b_ref[...][None, :]
    for s in range(SUB):
        rows = pl.ds(s * SUB_M, SUB_M)
        logits = jax.lax.dot_general(
            x_ref[rows, :], w, (((1,), (1,)), ((), ())),
            preferred_element_type=jnp.float32) + bias
        m = jnp.max(logits, axis=1, keepdims=True)
        e = jnp.exp(logits - m)
        o_ref[rows, :] = e / jnp.sum(e, axis=1, keepdims=True)


def kernel(x, W, b):
    grid = (TOKENS // TILE_M,)
    return pl.pallas_call(
        _gate_softmax_kernel,
        grid=grid,
        in_specs=[
            pl.BlockSpec((TILE_M, IN_CHANNELS), lambda i: (i, 0)),
            pl.BlockSpec((NUM_EXPERTS, IN_CHANNELS), lambda i: (0, 0)),
            pl.BlockSpec((NUM_EXPERTS,), lambda i: (0,)),
        ],
        out_specs=pl.BlockSpec((TILE_M, NUM_EXPERTS), lambda i: (i, 0)),
        out_shape=jax.ShapeDtypeStruct((TOKENS, NUM_EXPERTS), jnp.float32),
        compiler_params=pltpu.CompilerParams(
            dimension_semantics=("parallel",),
        ),
    )(x, W, b)


# speedup vs baseline: 1.0476x; 1.0408x over previous
import jax
import jax.numpy as jnp
from jax.experimental import pallas as pl
from jax.experimental.pallas import tpu as pltpu

TOKENS = 8192
IN_CHANNELS = 2048
NUM_EXPERTS = 64
TILE_M = 1024


def _gate_softmax_kernel(x_ref, w_ref, b_ref, o_ref):
    logits = jax.lax.dot_general(
        x_ref[...], w_ref[...], (((1,), (1,)), ((), ())),
        preferred_element_type=jnp.float32) + b_ref[...][None, :]
    m = jnp.max(logits, axis=1, keepdims=True)
    e = jnp.exp(logits - m)
    w = e / jnp.sum(e, axis=1, keepdims=True)
    o_ref[...] = w.reshape(TILE_M // 32, 32, NUM_EXPERTS)


def kernel(x, W, b):
    grid = (TOKENS // TILE_M,)
    return pl.pallas_call(
        _gate_softmax_kernel,
        grid=grid,
        in_specs=[
            pl.BlockSpec((TILE_M, IN_CHANNELS), lambda i: (i, 0)),
            pl.BlockSpec((NUM_EXPERTS, IN_CHANNELS), lambda i: (0, 0)),
            pl.BlockSpec((NUM_EXPERTS,), lambda i: (0,)),
        ],
        out_specs=pl.BlockSpec((TILE_M // 32, 32, NUM_EXPERTS),
                               lambda i: (i, 0, 0)),
        out_shape=jax.ShapeDtypeStruct((TOKENS // 32, 32, NUM_EXPERTS),
                                       jnp.float32),
        compiler_params=pltpu.CompilerParams(
            dimension_semantics=("parallel",),
        ),
    )(x, W, b).reshape(TOKENS, NUM_EXPERTS)


# speedup vs baseline: 1.1866x; 1.1327x over previous
import jax
import jax.numpy as jnp
from jax.experimental import pallas as pl
from jax.experimental.pallas import tpu as pltpu

TOKENS = 8192
IN_CHANNELS = 2048
NUM_EXPERTS = 64
TILE_M = 1024


def _gate_softmax_kernel(x_ref, w_ref, b_ref, o_ref):
    logits = jax.lax.dot_general(
        w_ref[...], x_ref[...], (((1,), (1,)), ((), ())),
        preferred_element_type=jnp.float32) + b_ref[...].reshape(NUM_EXPERTS, 1)
    m = jnp.max(logits, axis=0, keepdims=True)
    e = jnp.exp(logits - m)
    o_ref[...] = e / jnp.sum(e, axis=0, keepdims=True)


def kernel(x, W, b):
    grid = (TOKENS // TILE_M,)
    out = pl.pallas_call(
        _gate_softmax_kernel,
        grid=grid,
        in_specs=[
            pl.BlockSpec((TILE_M, IN_CHANNELS), lambda i: (i, 0)),
            pl.BlockSpec((NUM_EXPERTS, IN_CHANNELS), lambda i: (0, 0)),
            pl.BlockSpec((NUM_EXPERTS,), lambda i: (0,)),
        ],
        out_specs=pl.BlockSpec((NUM_EXPERTS, TILE_M), lambda i: (0, i)),
        out_shape=jax.ShapeDtypeStruct((NUM_EXPERTS, TOKENS), jnp.float32),
        compiler_params=pltpu.CompilerParams(
            dimension_semantics=("parallel",),
        ),
    )(x, W, b)
    return out.T
